# baseline (speedup 1.0000x reference)
import jax
import jax.numpy as jnp
from jax.experimental import pallas as pl
from jax.experimental.pallas import tpu as pltpu

_GRID = 50
_APL = 8
_TOTAL = _GRID * _GRID * _APL
_PRE = 6000
_POST = 1500
_THR = 0.7
_N = 6144
_C = 512
_K = _N // _C
_R = 1536


def _nms_kernel(d_ref, a_ref, s_ref, outb_ref, outs_ref, alive_ref, rows_ref):
    outb_ref[...] = jnp.zeros(outb_ref.shape, jnp.float32)
    outs_ref[...] = jnp.zeros(outs_ref.shape, jnp.float32)

    i0 = jax.lax.broadcasted_iota(jnp.int32, (_C, _C), 0)
    i1 = jax.lax.broadcasted_iota(jnp.int32, (_C, _C), 1)
    eye = jnp.where(i0 == i1, 1.0, 0.0)
    tri = jnp.where(i1 < i0, 1.0, 0.0)
    le = jnp.where(i0 <= i1, 1.0, 0.0)
    rlane = jax.lax.broadcasted_iota(jnp.int32, (1, _R), 1)

    def to_col(row):
        return jnp.sum(eye * row, axis=1, keepdims=True)

    def to_row(col):
        return jnp.sum(eye * col, axis=0, keepdims=True)

    def bslice(r, k):
        return rows_ref[r:r + 1, pl.ds(k * _C, _C)]

    def decode_block(k):
        base = k * _C
        a0 = a_ref[0:1, pl.ds(base, _C), 0:1].reshape(_C, 1)
        a1 = a_ref[0:1, pl.ds(base, _C), 1:2].reshape(_C, 1)
        a2 = a_ref[0:1, pl.ds(base, _C), 2:3].reshape(_C, 1)
        a3 = a_ref[0:1, pl.ds(base, _C), 3:4].reshape(_C, 1)
        anc_w = a3 - a1
        anc_h = a2 - a0
        anc_cx = a1 + 0.5 * anc_w
        anc_cy = a0 + 0.5 * anc_h
        d0 = d_ref[0:1, pl.ds(base, _C), 0:1].reshape(_C, 1) * 0.1
        d1 = d_ref[0:1, pl.ds(base, _C), 1:2].reshape(_C, 1) * 0.1
        d2 = d_ref[0:1, pl.ds(base, _C), 2:3].reshape(_C, 1) * 0.2
        d3 = d_ref[0:1, pl.ds(base, _C), 3:4].reshape(_C, 1) * 0.2
        bb_w = jnp.exp(d3) * anc_w
        bb_h = jnp.exp(d2) * anc_h
        bb_cx = d1 * anc_w + anc_cx
        bb_cy = d0 * anc_h + anc_cy
        y1 = bb_cy - 0.5 * bb_h
        x1 = bb_cx - 0.5 * bb_w
        y2 = y1 + bb_h
        x2 = x1 + bb_w
        area = (y2 - y1) * (x2 - x1)
        return y1, x1, y2, x2, area

    def stage_block(k, _):
        vals = decode_block(k)
        for r, v in enumerate(vals):
            rows_ref[r:r + 1, pl.ds(k * _C, _C)] = to_row(v)
        return 0

    jax.lax.fori_loop(0, _K, stage_block, 0)
    rows_ref[5:6, :] = s_ref[0]

    def iou_mask(cy1, cx1, cy2, cx2, car, py1, px1, py2, px2, par):
        iy1 = jnp.maximum(py1, cy1)
        ix1 = jnp.maximum(px1, cx1)
        iy2 = jnp.minimum(py2, cy2)
        ix2 = jnp.minimum(px2, cx2)
        inter = jnp.maximum(iy2 - iy1, 0.0) * jnp.maximum(ix2 - ix1, 0.0)
        iou = inter / (par + car - inter + 1e-8)
        return jnp.where(iou > _THR, 1.0, 0.0)

    def body_k(k, off):
        ry1, rx1 = bslice(0, k), bslice(1, k)
        ry2, rx2 = bslice(2, k), bslice(3, k)
        rar, rsc = bslice(4, k), bslice(5, k)
        cy1, cx1 = to_col(ry1), to_col(rx1)
        cy2, cx2 = to_col(ry2), to_col(rx2)
        car, csc = to_col(rar), to_col(rsc)
        gidx = k * _C + jax.lax.broadcasted_iota(jnp.int32, (_C, 1), 0)
        valid = jnp.where(gidx < _PRE, 1.0, 0.0)

        def body_p(p, supp):
            py1, px1 = bslice(0, p), bslice(1, p)
            py2, px2 = bslice(2, p), bslice(3, p)
            par = bslice(4, p)
            alive_p = alive_ref[pl.ds(p, 1), 0:1, :].reshape(1, _C)
            m = iou_mask(cy1, cx1, cy2, cx2, car, py1, px1, py2, px2, par)
            return jnp.maximum(supp,
                               jnp.max(m * alive_p, axis=1, keepdims=True))

        supp = jax.lax.fori_loop(0, k, body_p, jnp.zeros((_C, 1), jnp.float32))
        cand = valid * (1.0 - supp)

        md = iou_mask(cy1, cx1, cy2, cx2, car,
                      ry1, rx1, ry2, rx2, rar) * tri

        def fix_cond(st):
            it, ch, _ = st
            return (ch > 0.0) & (it < _C)

        def fix_body(st):
            it, _, srow = st
            kill = jnp.max(md * srow, axis=1, keepdims=True)
            scol = cand * (1.0 - kill)
            srow2 = jnp.sum(eye * scol, axis=0, keepdims=True)
            ch = jnp.max(jnp.abs(srow2 - srow))
            return it + 1, ch, srow2

        srow0 = jnp.sum(eye * cand, axis=0, keepdims=True)
        _, _, srow = jax.lax.while_loop(
            fix_cond, fix_body, (jnp.int32(0), jnp.float32(1.0), srow0))
        scol = cand * (1.0 - jnp.max(md * srow, axis=1, keepdims=True))

        alive_ref[pl.ds(k, 1), 0:1, :] = srow.reshape(1, 1, _C)

        incl = jnp.sum(le * scol, axis=0, keepdims=True)
        cnt = jnp.sum(scol)
        rank_row = jnp.where(srow > 0.5, off + incl - 1.0, -1.0)
        rank_col = to_col(rank_row)
        onehot = jnp.where(rank_col.astype(jnp.int32) == rlane, 1.0, 0.0)

        ccy1 = jnp.clip(cy1, 0.0, 1.0)
        ccx1 = jnp.clip(cx1, 0.0, 1.0)
        ccy2 = jnp.clip(cy2, 0.0, 1.0)
        ccx2 = jnp.clip(cx2, 0.0, 1.0)
        for c, v in enumerate((ccy1, ccx1, ccy2, ccx2)):
            outb_ref[0:1, c:c + 1, :] = outb_ref[0:1, c:c + 1, :] + jnp.sum(
                onehot * v, axis=0, keepdims=True).reshape(1, 1, _R)
        outs_ref[0:1, 0:1, :] = outs_ref[0:1, 0:1, :] + jnp.sum(
            onehot * csc, axis=0, keepdims=True).reshape(1, 1, _R)
        return off + cnt

    def outer_cond(st):
        k, off = st
        return (k < _K) & (off < float(_POST))

    def outer_body(st):
        k, off = st
        return k + 1, body_k(k, off)

    jax.lax.while_loop(outer_cond, outer_body, (jnp.int32(0), jnp.float32(0.0)))


def kernel(rpn_bbox_deltas, rpn_labels, anchors):
    b = rpn_bbox_deltas.shape[0]
    deltas = rpn_bbox_deltas.reshape(b, _TOTAL, 4)
    scores = jax.nn.softmax(rpn_labels, axis=-1).reshape(b, _TOTAL)
    ssc, idx = jax.lax.top_k(scores, _PRE)
    pad = _N - _PRE
    idx_p = jnp.pad(idx, ((0, 0), (0, pad)))
    gd = jnp.take_along_axis(deltas, idx_p[..., None], axis=1)
    ga = anchors[idx_p]
    ssc_p = jnp.pad(ssc, ((0, 0), (0, pad)))
    s_row = ssc_p[:, None, :]

    outb, outs = pl.pallas_call(
        _nms_kernel,
        grid=(b,),
        in_specs=[pl.BlockSpec((1, _N, 4), lambda i: (i, 0, 0)),
                  pl.BlockSpec((1, _N, 4), lambda i: (i, 0, 0)),
                  pl.BlockSpec((1, 1, _N), lambda i: (i, 0, 0))],
        out_specs=[pl.BlockSpec((1, 4, _R), lambda i: (i, 0, 0)),
                   pl.BlockSpec((1, 1, _R), lambda i: (i, 0, 0))],
        out_shape=[jax.ShapeDtypeStruct((b, 4, _R), jnp.float32),
                   jax.ShapeDtypeStruct((b, 1, _R), jnp.float32)],
        scratch_shapes=[pltpu.VMEM((_K, 8, _C), jnp.float32),
                        pltpu.VMEM((8, _N), jnp.float32)],
    )(gd, ga, s_row)

    roi_bboxes = outb[:, :, :_POST].transpose(0, 2, 1)
    roi_scores = outs[:, 0, :_POST]
    return roi_bboxes, roi_scores


# speedup vs baseline: 1.2445x; 1.2445x over previous
import jax
import jax.numpy as jnp
from jax.experimental import pallas as pl
from jax.experimental.pallas import tpu as pltpu

_GRID = 50
_APL = 8
_TOTAL = _GRID * _GRID * _APL
_PRE = 6000
_POST = 1500
_THR = 0.7
_N = 6144
_C = 512
_K = _N // _C
_R = 1536


def _nms_kernel(d_ref, a_ref, s_ref, outb_ref, outs_ref, alive_ref, rows_ref):
    d = d_ref[0]
    a = a_ref[0]
    sc = s_ref[0]

    a0, a1, a2, a3 = a[0:1, :], a[1:2, :], a[2:3, :], a[3:4, :]
    anc_w = a3 - a1
    anc_h = a2 - a0
    anc_cx = a1 + 0.5 * anc_w
    anc_cy = a0 + 0.5 * anc_h
    d0 = d[0:1, :] * 0.1
    d1 = d[1:2, :] * 0.1
    d2 = d[2:3, :] * 0.2
    d3 = d[3:4, :] * 0.2
    bb_w = jnp.exp(d3) * anc_w
    bb_h = jnp.exp(d2) * anc_h
    bb_cx = d1 * anc_w + anc_cx
    bb_cy = d0 * anc_h + anc_cy
    y1 = bb_cy - 0.5 * bb_h
    x1 = bb_cx - 0.5 * bb_w
    y2 = y1 + bb_h
    x2 = x1 + bb_w
    area = (y2 - y1) * (x2 - x1)

    rows_ref[0:1, :] = y1
    rows_ref[1:2, :] = x1
    rows_ref[2:3, :] = y2
    rows_ref[3:4, :] = x2
    rows_ref[4:5, :] = area
    rows_ref[5:6, :] = sc

    outb_ref[...] = jnp.zeros(outb_ref.shape, jnp.float32)
    outs_ref[...] = jnp.zeros(outs_ref.shape, jnp.float32)

    i0 = jax.lax.broadcasted_iota(jnp.int32, (_C, _C), 0)
    i1 = jax.lax.broadcasted_iota(jnp.int32, (_C, _C), 1)
    eye = jnp.where(i0 == i1, 1.0, 0.0)
    tri = jnp.where(i1 < i0, 1.0, 0.0)
    le = jnp.where(i0 <= i1, 1.0, 0.0)
    rlane = jax.lax.broadcasted_iota(jnp.int32, (1, _R), 1)

    def to_col(row):
        return jnp.sum(eye * row, axis=1, keepdims=True)

    def bslice(r, k):
        return rows_ref[r:r + 1, pl.ds(k * _C, _C)]

    def iou_mask(cy1, cx1, cy2, cx2, car, py1, px1, py2, px2, par):
        iy1 = jnp.maximum(py1, cy1)
        ix1 = jnp.maximum(px1, cx1)
        iy2 = jnp.minimum(py2, cy2)
        ix2 = jnp.minimum(px2, cx2)
        inter = jnp.maximum(iy2 - iy1, 0.0) * jnp.maximum(ix2 - ix1, 0.0)
        iou = inter / (par + car - inter + 1e-8)
        return jnp.where(iou > _THR, 1.0, 0.0)

    def body_k(k, off):
        ry1, rx1 = bslice(0, k), bslice(1, k)
        ry2, rx2 = bslice(2, k), bslice(3, k)
        rar, rsc = bslice(4, k), bslice(5, k)
        cy1, cx1 = to_col(ry1), to_col(rx1)
        cy2, cx2 = to_col(ry2), to_col(rx2)
        car, csc = to_col(rar), to_col(rsc)
        gidx = k * _C + jax.lax.broadcasted_iota(jnp.int32, (_C, 1), 0)
        valid = jnp.where(gidx < _PRE, 1.0, 0.0)

        def body_p(p, supp):
            py1, px1 = bslice(0, p), bslice(1, p)
            py2, px2 = bslice(2, p), bslice(3, p)
            par = bslice(4, p)
            alive_p = alive_ref[pl.ds(p, 1), 0:1, :].reshape(1, _C)
            m = iou_mask(cy1, cx1, cy2, cx2, car, py1, px1, py2, px2, par)
            return jnp.maximum(supp,
                               jnp.max(m * alive_p, axis=1, keepdims=True))

        supp = jax.lax.fori_loop(0, k, body_p, jnp.zeros((_C, 1), jnp.float32))
        cand = valid * (1.0 - supp)

        md = iou_mask(cy1, cx1, cy2, cx2, car,
                      ry1, rx1, ry2, rx2, rar) * tri

        def fix_cond(st):
            it, ch, _ = st
            return (ch > 0.0) & (it < _C)

        def fix_body(st):
            it, _, srow = st
            kill = jnp.max(md * srow, axis=1, keepdims=True)
            scol = cand * (1.0 - kill)
            srow2 = jnp.sum(eye * scol, axis=0, keepdims=True)
            ch = jnp.max(jnp.abs(srow2 - srow))
            return it + 1, ch, srow2

        srow0 = jnp.sum(eye * cand, axis=0, keepdims=True)
        _, _, srow = jax.lax.while_loop(
            fix_cond, fix_body, (jnp.int32(0), jnp.float32(1.0), srow0))
        scol = cand * (1.0 - jnp.max(md * srow, axis=1, keepdims=True))

        alive_ref[pl.ds(k, 1), 0:1, :] = srow.reshape(1, 1, _C)

        incl = jnp.sum(le * scol, axis=0, keepdims=True)
        cnt = jnp.sum(scol)
        rank_row = jnp.where(srow > 0.5, off + incl - 1.0, -1.0)
        rank_col = to_col(rank_row)
        onehot = jnp.where(rank_col.astype(jnp.int32) == rlane, 1.0, 0.0)

        ccy1 = jnp.clip(cy1, 0.0, 1.0)
        ccx1 = jnp.clip(cx1, 0.0, 1.0)
        ccy2 = jnp.clip(cy2, 0.0, 1.0)
        ccx2 = jnp.clip(cx2, 0.0, 1.0)
        for c, v in enumerate((ccy1, ccx1, ccy2, ccx2)):
            outb_ref[0:1, c:c + 1, :] = outb_ref[0:1, c:c + 1, :] + jnp.sum(
                onehot * v, axis=0, keepdims=True).reshape(1, 1, _R)
        outs_ref[0:1, 0:1, :] = outs_ref[0:1, 0:1, :] + jnp.sum(
            onehot * csc, axis=0, keepdims=True).reshape(1, 1, _R)
        return off + cnt

    def outer_cond(st):
        k, off = st
        return (k < _K) & (off < float(_POST))

    def outer_body(st):
        k, off = st
        return k + 1, body_k(k, off)

    jax.lax.while_loop(outer_cond, outer_body, (jnp.int32(0), jnp.float32(0.0)))


def kernel(rpn_bbox_deltas, rpn_labels, anchors):
    b = rpn_bbox_deltas.shape[0]
    deltas = rpn_bbox_deltas.reshape(b, _TOTAL, 4)
    scores = jax.nn.softmax(rpn_labels, axis=-1).reshape(b, _TOTAL)
    ssc, idx = jax.lax.top_k(scores, _PRE)
    pad = _N - _PRE
    idx_p = jnp.pad(idx, ((0, 0), (0, pad)))[:, None, :]
    deltas_t = deltas.transpose(0, 2, 1)
    anchors_t = jnp.broadcast_to(anchors.T[None], (b, 4, _TOTAL))
    d_row = jnp.take_along_axis(deltas_t, idx_p, axis=2)
    a_row = jnp.take_along_axis(anchors_t, idx_p, axis=2)
    ssc_p = jnp.pad(ssc, ((0, 0), (0, pad)))
    s_row = ssc_p[:, None, :]

    outb, outs = pl.pallas_call(
        _nms_kernel,
        grid=(b,),
        in_specs=[pl.BlockSpec((1, 4, _N), lambda i: (i, 0, 0)),
                  pl.BlockSpec((1, 4, _N), lambda i: (i, 0, 0)),
                  pl.BlockSpec((1, 1, _N), lambda i: (i, 0, 0))],
        out_specs=[pl.BlockSpec((1, 4, _R), lambda i: (i, 0, 0)),
                   pl.BlockSpec((1, 1, _R), lambda i: (i, 0, 0))],
        out_shape=[jax.ShapeDtypeStruct((b, 4, _R), jnp.float32),
                   jax.ShapeDtypeStruct((b, 1, _R), jnp.float32)],
        scratch_shapes=[pltpu.VMEM((_K, 8, _C), jnp.float32),
                        pltpu.VMEM((8, _N), jnp.float32)],
    )(d_row, a_row, s_row)

    roi_bboxes = outb[:, :, :_POST].transpose(0, 2, 1)
    roi_scores = outs[:, 0, :_POST]
    return roi_bboxes, roi_scores
